# baseline (speedup 1.0000x reference)
import functools

import jax
import jax.numpy as jnp
from jax import lax
from jax.experimental import pallas as pl
from jax.experimental.pallas import tpu as pltpu
from jax.experimental.pallas import tpu_sc as plsc

NC = 2
NS = 16
NW = NC * NS
CH = 128

F32 = jnp.float32


def _cdiv(a, b):
    return (a + b - 1) // b


def _sc_gather(table, idx):
    T, D = table.shape
    B = idx.shape[0]
    nchunk = B // CH
    iters = _cdiv(nchunk, NW)
    mesh = plsc.VectorSubcoreMesh(core_axis_name="c", subcore_axis_name="s")

    @functools.partial(
        pl.kernel,
        out_type=jax.ShapeDtypeStruct((B, D), F32),
        mesh=mesh,
        scratch_types=[
            tuple(pltpu.VMEM((CH,), jnp.int32) for _ in range(1)),
            tuple(pltpu.VMEM((CH, D), F32) for _ in range(1)),
            pltpu.SemaphoreType.DMA,
            pltpu.SemaphoreType.DMA,
            pltpu.SemaphoreType.DMA,
        ],
    )
    def k(table_hbm, idx_hbm, out_hbm, idx_vs, rows_vs, sem, sem2, sem3):
        idx_v, rows_v = idx_vs[0], rows_vs[0]
        wid = lax.axis_index("s") * NC + lax.axis_index("c")

        @pl.loop(0, iters)
        def _(i):
            chunk = i * NW + wid

            @pl.when(chunk < nchunk)
            def _():
                base = pl.multiple_of(chunk * CH, CH)
                pltpu.sync_copy(idx_hbm.at[pl.ds(base, CH)], idx_v)
                pltpu.async_copy(table_hbm.at[idx_v], rows_v, sem).wait()
                pltpu.sync_copy(rows_v, out_hbm.at[pl.ds(base, CH)])

    return k(table, idx)


def _sc_scatter_add(rows_table, src, dst, zeros_n, gather_rows):
    T, D = rows_table.shape
    E = dst.shape[0]
    N = zeros_n.shape[0]
    nzc = 1000
    nz_tiles = N // nzc
    nchunk = E // CH
    per_core = nchunk // NC
    iters = _cdiv(per_core, NS)
    mesh = plsc.VectorSubcoreMesh(core_axis_name="c", subcore_axis_name="s")

    @functools.partial(
        pl.kernel,
        out_type=jax.ShapeDtypeStruct((NC * N, D), F32),
        mesh=mesh,
        scratch_types=[
            pltpu.VMEM((CH,), jnp.int32),
            pltpu.VMEM((CH,), jnp.int32),
            pltpu.VMEM((CH, D), F32),
            pltpu.VMEM_SHARED((N + 8, D), F32),
            pltpu.SemaphoreType.DMA,
        ],
    )
    def k(table_hbm, src_hbm, dst_hbm, zeros_hbm, out_hbm,
          sidx_v, didx_v, rows_v, acc_sh, sem):
        c = lax.axis_index("c")
        s = lax.axis_index("s")

        row0 = pl.multiple_of(s * nzc, 8)

        @pl.when(s < nz_tiles)
        def _():
            pltpu.sync_copy(zeros_hbm.at[pl.ds(row0, nzc)],
                            acc_sh.at[pl.ds(row0, nzc)])

        plsc.subcore_barrier()

        @pl.loop(0, iters)
        def _(i):
            local = i * NS + s

            @pl.when(local < per_core)
            def _():
                chunk = c * per_core + local
                base = pl.multiple_of(chunk * CH, CH)
                pltpu.sync_copy(dst_hbm.at[pl.ds(base, CH)], didx_v)
                if gather_rows:
                    pltpu.sync_copy(src_hbm.at[pl.ds(base, CH)], sidx_v)
                    pltpu.async_copy(table_hbm.at[sidx_v], rows_v, sem).wait()
                else:
                    pltpu.sync_copy(table_hbm.at[pl.ds(base, CH)], rows_v)
                pltpu.sync_copy(rows_v, acc_sh.at[didx_v], add=True)

        plsc.subcore_barrier()

        @pl.when(s < nz_tiles)
        def _():
            out0 = pl.multiple_of(c * N + s * nzc, 8)
            pltpu.sync_copy(acc_sh.at[pl.ds(row0, nzc)],
                            out_hbm.at[pl.ds(out0, nzc)])

    return k(rows_table, src, dst, zeros_n)


def _tc_matmul(a, w, b):
    M, K = a.shape
    Dout = w.shape[1]
    BM = 2000 if M % 2000 == 0 else M

    def body(a_ref, w_ref, b_ref, o_ref):
        o_ref[...] = jnp.dot(a_ref[...], w_ref[...],
                             preferred_element_type=F32) + b_ref[...]

    return pl.pallas_call(
        body,
        grid=(M // BM,),
        in_specs=[
            pl.BlockSpec((BM, K), lambda i: (i, 0)),
            pl.BlockSpec((K, Dout), lambda i: (0, 0)),
            pl.BlockSpec((1, Dout), lambda i: (0, 0)),
        ],
        out_specs=pl.BlockSpec((BM, Dout), lambda i: (i, 0)),
        out_shape=jax.ShapeDtypeStruct((M, Dout), F32),
    )(a, w, b)


def _tc_input_layer(xws, ea, wieT, bi):
    E, D = xws.shape
    DE = ea.shape[1]
    BE = 2048

    def body(xws_ref, ea_ref, w_ref, b_ref, h0_ref, hc_ref):
        h0 = xws_ref[...] + jnp.dot(ea_ref[...], w_ref[...],
                                    preferred_element_type=F32) + b_ref[...]
        h0_ref[...] = h0
        hc_ref[...] = jnp.maximum(h0, 0.0)

    return pl.pallas_call(
        body,
        grid=(E // BE,),
        in_specs=[
            pl.BlockSpec((BE, D), lambda i: (i, 0)),
            pl.BlockSpec((BE, DE), lambda i: (i, 0)),
            pl.BlockSpec((DE, D), lambda i: (0, 0)),
            pl.BlockSpec((1, D), lambda i: (0, 0)),
        ],
        out_specs=[
            pl.BlockSpec((BE, D), lambda i: (i, 0)),
            pl.BlockSpec((BE, D), lambda i: (i, 0)),
        ],
        out_shape=[
            jax.ShapeDtypeStruct((E, D), F32),
            jax.ShapeDtypeStruct((E, D), F32),
        ],
    )(xws, ea, wieT, bi)


def _tc_psum(p, n):
    D = p.shape[1]
    BN = 2000
    nb = n // BN

    def body(a_ref, b_ref, o_ref):
        o_ref[...] = a_ref[...] + b_ref[...]

    return pl.pallas_call(
        body,
        grid=(nb,),
        in_specs=[
            pl.BlockSpec((BN, D), lambda i: (i, 0)),
            pl.BlockSpec((BN, D), lambda i: (i + nb, 0)),
        ],
        out_specs=pl.BlockSpec((BN, D), lambda i: (i, 0)),
        out_shape=jax.ShapeDtypeStruct((n, D), F32),
    )(p, p)


def _tc_round(h0, arows, rrows, whT, bh):
    E, D = h0.shape
    BE = 2048

    def body(h0_ref, a_ref, r_ref, w_ref, b_ref, o_ref):
        m = a_ref[...] - r_ref[...]
        o_ref[...] = jnp.maximum(
            h0_ref[...] + jnp.dot(m, w_ref[...], preferred_element_type=F32)
            + b_ref[...], 0.0)

    return pl.pallas_call(
        body,
        grid=(E // BE,),
        in_specs=[
            pl.BlockSpec((BE, D), lambda i: (i, 0)),
            pl.BlockSpec((BE, D), lambda i: (i, 0)),
            pl.BlockSpec((BE, D), lambda i: (i, 0)),
            pl.BlockSpec((D, D), lambda i: (0, 0)),
            pl.BlockSpec((1, D), lambda i: (0, 0)),
        ],
        out_specs=pl.BlockSpec((BE, D), lambda i: (i, 0)),
        out_shape=jax.ShapeDtypeStruct((E, D), F32),
    )(h0, arows, rrows, whT, bh)


def _tc_readout(x, mparts, woxT, womT, bo, n):
    D = x.shape[1]
    BN = 2000
    nb = n // BN

    def body(x_ref, p0_ref, p1_ref, wx_ref, wm_ref, b_ref, o_ref):
        mn = p0_ref[...] + p1_ref[...]
        iso = jnp.sum(mn, axis=1, keepdims=True) == 0.0
        mne = jnp.where(iso, x_ref[...], mn)
        o_ref[...] = jnp.maximum(
            jnp.dot(x_ref[...], wx_ref[...], preferred_element_type=F32)
            + jnp.dot(mne, wm_ref[...], preferred_element_type=F32)
            + b_ref[...], 0.0)

    return pl.pallas_call(
        body,
        grid=(nb,),
        in_specs=[
            pl.BlockSpec((BN, D), lambda i: (i, 0)),
            pl.BlockSpec((BN, D), lambda i: (i, 0)),
            pl.BlockSpec((BN, D), lambda i: (i + nb, 0)),
            pl.BlockSpec((D, D), lambda i: (0, 0)),
            pl.BlockSpec((D, D), lambda i: (0, 0)),
            pl.BlockSpec((1, D), lambda i: (0, 0)),
        ],
        out_specs=pl.BlockSpec((BN, D), lambda i: (i, 0)),
        out_shape=jax.ShapeDtypeStruct((n, D), F32),
    )(x, mparts, mparts, woxT, womT, bo)


def kernel(x, edge_index, edge_attr, rev_edge_index, W_i, b_i, W_h, b_h,
           W_o, b_o):
    n, df = x.shape
    e = edge_index.shape[1]
    de = edge_attr.shape[1]
    hid = W_i.shape[0]
    depth = 5

    e_pad = 327680
    pad = e_pad - e
    src = jnp.concatenate([edge_index[0].astype(jnp.int32),
                           jnp.zeros((pad,), jnp.int32)])
    dst = jnp.concatenate([edge_index[1].astype(jnp.int32),
                           jnp.full((pad,), n, jnp.int32)])
    rev = jnp.concatenate([rev_edge_index.astype(jnp.int32),
                           jnp.zeros((pad,), jnp.int32)])
    ea = jnp.concatenate([edge_attr, jnp.zeros((pad, de), F32)])

    wixT = W_i[:, :df].T
    wieT = W_i[:, df:].T
    whT = W_h.T
    woxT = W_o[:, :df].T
    womT = W_o[:, df:].T
    bi = b_i.reshape(1, hid)
    bh = b_h.reshape(1, hid)
    bo = b_o.reshape(1, hid)
    zeros_n = jnp.zeros((n, hid), dtype=F32)

    xw = _tc_matmul(x, wixT, jnp.zeros((1, hid), dtype=F32))
    xws = _sc_gather(xw, src)
    h0, hc = _tc_input_layer(xws, ea, wieT, bi)

    for _ in range(1, depth):
        aparts = _sc_scatter_add(hc, src, dst, zeros_n, gather_rows=True)
        a = _tc_psum(aparts, n)
        arows = _sc_gather(a, src)
        rrows = _sc_gather(hc, rev)
        hc = _tc_round(h0, arows, rrows, whT, bh)

    mparts = _sc_scatter_add(hc, src, dst, zeros_n, gather_rows=False)
    return _tc_readout(x, mparts, woxT, womT, bo, n)


# speedup vs baseline: 2.4763x; 2.4763x over previous
import functools

import jax
import jax.numpy as jnp
from jax import lax
from jax.experimental import pallas as pl
from jax.experimental.pallas import tpu as pltpu
from jax.experimental.pallas import tpu_sc as plsc

NC = 2
NS = 16
NW = NC * NS
CH = 128
K = 3

F32 = jnp.float32


def _cdiv(a, b):
    return (a + b - 1) // b


def _sc_gather(table, idx):
    T, D = table.shape
    B = idx.shape[0]
    nchunk = B // CH
    bulk = (nchunk // NW) // K * K
    groups = bulk // K
    ntail = nchunk - bulk * NW
    assert ntail <= NW
    mesh = plsc.VectorSubcoreMesh(core_axis_name="c", subcore_axis_name="s")

    @functools.partial(
        pl.kernel,
        out_type=jax.ShapeDtypeStruct((B, D), F32),
        mesh=mesh,
        scratch_types=[
            tuple(pltpu.VMEM((CH,), jnp.int32) for _ in range(K)),
            tuple(pltpu.VMEM((CH, D), F32) for _ in range(K)),
            pltpu.SemaphoreType.DMA,
            pltpu.SemaphoreType.DMA,
            pltpu.SemaphoreType.DMA,
        ],
    )
    def k(table_hbm, idx_hbm, out_hbm, idx_bufs, row_bufs, sem_i, sem_g, sem_o):
        wid = lax.axis_index("s") * NC + lax.axis_index("c")

        @pl.loop(0, groups)
        def _(g):
            bases = [pl.multiple_of(((g * K + b) * NW + wid) * CH, CH)
                     for b in range(K)]
            ds = [pltpu.async_copy(idx_hbm.at[pl.ds(bases[b], CH)],
                                   idx_bufs[b], sem_i) for b in range(K)]
            for d in ds:
                d.wait()
            ds = [pltpu.async_copy(table_hbm.at[idx_bufs[b]], row_bufs[b],
                                   sem_g) for b in range(K)]
            for d in ds:
                d.wait()
            ds = [pltpu.async_copy(row_bufs[b], out_hbm.at[pl.ds(bases[b], CH)],
                                   sem_o) for b in range(K)]
            for d in ds:
                d.wait()

        @pl.when(wid < ntail)
        def _():
            base = pl.multiple_of((bulk * NW + wid) * CH, CH)
            pltpu.sync_copy(idx_hbm.at[pl.ds(base, CH)], idx_bufs[0])
            pltpu.async_copy(table_hbm.at[idx_bufs[0]], row_bufs[0],
                             sem_g).wait()
            pltpu.sync_copy(row_bufs[0], out_hbm.at[pl.ds(base, CH)])

    return k(table, idx)


def _sc_dual_gather(a, hc, src, rev):
    N, D = a.shape
    E = src.shape[0]
    nchunk = E // CH
    bulk = (nchunk // NW) // K * K
    groups = bulk // K
    ntail = nchunk - bulk * NW
    assert ntail <= NW
    mesh = plsc.VectorSubcoreMesh(core_axis_name="c", subcore_axis_name="s")

    @functools.partial(
        pl.kernel,
        out_type=(jax.ShapeDtypeStruct((E, D), F32),
                  jax.ShapeDtypeStruct((E, D), F32)),
        mesh=mesh,
        scratch_types=[
            tuple(pltpu.VMEM((CH,), jnp.int32) for _ in range(K)),
            tuple(pltpu.VMEM((CH,), jnp.int32) for _ in range(K)),
            tuple(pltpu.VMEM((CH, D), F32) for _ in range(K)),
            tuple(pltpu.VMEM((CH, D), F32) for _ in range(K)),
            pltpu.SemaphoreType.DMA,
            pltpu.SemaphoreType.DMA,
            pltpu.SemaphoreType.DMA,
        ],
    )
    def k(a_hbm, hc_hbm, src_hbm, rev_hbm, ar_hbm, rr_hbm,
          aidx_bufs, ridx_bufs, ar_bufs, rr_bufs, sem_i, sem_g, sem_o):
        wid = lax.axis_index("s") * NC + lax.axis_index("c")

        @pl.loop(0, groups)
        def _(g):
            bases = [pl.multiple_of(((g * K + b) * NW + wid) * CH, CH)
                     for b in range(K)]
            ds = []
            for b in range(K):
                ds.append(pltpu.async_copy(src_hbm.at[pl.ds(bases[b], CH)],
                                           aidx_bufs[b], sem_i))
                ds.append(pltpu.async_copy(rev_hbm.at[pl.ds(bases[b], CH)],
                                           ridx_bufs[b], sem_i))
            for d in ds:
                d.wait()
            ds = []
            for b in range(K):
                ds.append(pltpu.async_copy(a_hbm.at[aidx_bufs[b]], ar_bufs[b],
                                           sem_g))
                ds.append(pltpu.async_copy(hc_hbm.at[ridx_bufs[b]], rr_bufs[b],
                                           sem_g))
            for d in ds:
                d.wait()
            ds = []
            for b in range(K):
                ds.append(pltpu.async_copy(ar_bufs[b],
                                           ar_hbm.at[pl.ds(bases[b], CH)],
                                           sem_o))
                ds.append(pltpu.async_copy(rr_bufs[b],
                                           rr_hbm.at[pl.ds(bases[b], CH)],
                                           sem_o))
            for d in ds:
                d.wait()

        @pl.when(wid < ntail)
        def _():
            base = pl.multiple_of((bulk * NW + wid) * CH, CH)
            pltpu.sync_copy(src_hbm.at[pl.ds(base, CH)], aidx_bufs[0])
            pltpu.sync_copy(rev_hbm.at[pl.ds(base, CH)], ridx_bufs[0])
            da = pltpu.async_copy(a_hbm.at[aidx_bufs[0]], ar_bufs[0], sem_g)
            dr = pltpu.async_copy(hc_hbm.at[ridx_bufs[0]], rr_bufs[0], sem_g)
            da.wait()
            dr.wait()
            pltpu.sync_copy(ar_bufs[0], ar_hbm.at[pl.ds(base, CH)])
            pltpu.sync_copy(rr_bufs[0], rr_hbm.at[pl.ds(base, CH)])

    return k(a, hc, src, rev)


def _sc_scatter_add(rows_table, src, dst, zeros_n, gather_rows):
    T, D = rows_table.shape
    E = dst.shape[0]
    N = zeros_n.shape[0]
    nzc = 1000
    nz_tiles = N // nzc
    nchunk = E // CH
    per_core = nchunk // NC
    bulk = (per_core // NS) // K * K
    groups = bulk // K
    ntail = per_core - bulk * NS
    assert ntail <= NS
    mesh = plsc.VectorSubcoreMesh(core_axis_name="c", subcore_axis_name="s")

    @functools.partial(
        pl.kernel,
        out_type=jax.ShapeDtypeStruct((NC * N, D), F32),
        mesh=mesh,
        scratch_types=[
            tuple(pltpu.VMEM((CH,), jnp.int32) for _ in range(K)),
            tuple(pltpu.VMEM((CH,), jnp.int32) for _ in range(K)),
            tuple(pltpu.VMEM((CH, D), F32) for _ in range(K)),
            pltpu.VMEM_SHARED((N, D), F32),
            pltpu.SemaphoreType.DMA,
            pltpu.SemaphoreType.DMA,
            pltpu.SemaphoreType.DMA,
        ],
    )
    def k(table_hbm, src_hbm, dst_hbm, zeros_hbm, out_hbm,
          sidx_bufs, didx_bufs, row_bufs, acc_sh, sem_i, sem_g, sem_a):
        c = lax.axis_index("c")
        s = lax.axis_index("s")

        row0 = pl.multiple_of(s * nzc, 8)

        @pl.when(s < nz_tiles)
        def _():
            pltpu.sync_copy(zeros_hbm.at[pl.ds(row0, nzc)],
                            acc_sh.at[pl.ds(row0, nzc)])

        plsc.subcore_barrier()

        @pl.loop(0, groups)
        def _(g):
            bases = [pl.multiple_of(
                (c * per_core + (g * K + b) * NS + s) * CH, CH)
                for b in range(K)]
            ds = []
            for b in range(K):
                ds.append(pltpu.async_copy(dst_hbm.at[pl.ds(bases[b], CH)],
                                           didx_bufs[b], sem_i))
                if gather_rows:
                    ds.append(pltpu.async_copy(src_hbm.at[pl.ds(bases[b], CH)],
                                               sidx_bufs[b], sem_i))
            for d in ds:
                d.wait()
            ds = []
            for b in range(K):
                if gather_rows:
                    ds.append(pltpu.async_copy(table_hbm.at[sidx_bufs[b]],
                                               row_bufs[b], sem_g))
                else:
                    ds.append(pltpu.async_copy(
                        table_hbm.at[pl.ds(bases[b], CH)], row_bufs[b], sem_g))
            for d in ds:
                d.wait()
            ds = [pltpu.async_copy(row_bufs[b], acc_sh.at[didx_bufs[b]],
                                   sem_a, add=True) for b in range(K)]
            for d in ds:
                d.wait()

        @pl.when(s < ntail)
        def _():
            base = pl.multiple_of((c * per_core + bulk * NS + s) * CH, CH)
            pltpu.sync_copy(dst_hbm.at[pl.ds(base, CH)], didx_bufs[0])
            if gather_rows:
                pltpu.sync_copy(src_hbm.at[pl.ds(base, CH)], sidx_bufs[0])
                pltpu.async_copy(table_hbm.at[sidx_bufs[0]], row_bufs[0],
                                 sem_g).wait()
            else:
                pltpu.sync_copy(table_hbm.at[pl.ds(base, CH)], row_bufs[0])
            pltpu.sync_copy(row_bufs[0], acc_sh.at[didx_bufs[0]], add=True)

        plsc.subcore_barrier()

        @pl.when(s < nz_tiles)
        def _():
            out0 = pl.multiple_of(c * N + s * nzc, 8)
            pltpu.sync_copy(acc_sh.at[pl.ds(row0, nzc)],
                            out_hbm.at[pl.ds(out0, nzc)])

    return k(rows_table, src, dst, zeros_n)


def _tc_matmul(a, w, b):
    M, Kd = a.shape
    Dout = w.shape[1]
    BM = 2000 if M % 2000 == 0 else M

    def body(a_ref, w_ref, b_ref, o_ref):
        o_ref[...] = jnp.dot(a_ref[...], w_ref[...],
                             preferred_element_type=F32) + b_ref[...]

    return pl.pallas_call(
        body,
        grid=(M // BM,),
        in_specs=[
            pl.BlockSpec((BM, Kd), lambda i: (i, 0)),
            pl.BlockSpec((Kd, Dout), lambda i: (0, 0)),
            pl.BlockSpec((1, Dout), lambda i: (0, 0)),
        ],
        out_specs=pl.BlockSpec((BM, Dout), lambda i: (i, 0)),
        out_shape=jax.ShapeDtypeStruct((M, Dout), F32),
    )(a, w, b)


def _tc_input_layer(xws, ea, wieT, bi):
    E, D = xws.shape
    DE = ea.shape[1]
    BE = 1600

    def body(xws_ref, ea_ref, w_ref, b_ref, h0_ref, hc_ref):
        h0 = xws_ref[...] + jnp.dot(ea_ref[...], w_ref[...],
                                    preferred_element_type=F32) + b_ref[...]
        h0_ref[...] = h0
        hc_ref[...] = jnp.maximum(h0, 0.0)

    return pl.pallas_call(
        body,
        grid=(E // BE,),
        in_specs=[
            pl.BlockSpec((BE, D), lambda i: (i, 0)),
            pl.BlockSpec((BE, DE), lambda i: (i, 0)),
            pl.BlockSpec((DE, D), lambda i: (0, 0)),
            pl.BlockSpec((1, D), lambda i: (0, 0)),
        ],
        out_specs=[
            pl.BlockSpec((BE, D), lambda i: (i, 0)),
            pl.BlockSpec((BE, D), lambda i: (i, 0)),
        ],
        out_shape=[
            jax.ShapeDtypeStruct((E, D), F32),
            jax.ShapeDtypeStruct((E, D), F32),
        ],
    )(xws, ea, wieT, bi)


def _tc_psum(p, n):
    D = p.shape[1]
    BN = 2000
    nb = n // BN

    def body(a_ref, b_ref, o_ref):
        o_ref[...] = a_ref[...] + b_ref[...]

    return pl.pallas_call(
        body,
        grid=(nb,),
        in_specs=[
            pl.BlockSpec((BN, D), lambda i: (i, 0)),
            pl.BlockSpec((BN, D), lambda i: (i + nb, 0)),
        ],
        out_specs=pl.BlockSpec((BN, D), lambda i: (i, 0)),
        out_shape=jax.ShapeDtypeStruct((n, D), F32),
    )(p, p)


def _tc_round(h0, arows, rrows, whT, bh):
    E, D = h0.shape
    BE = 1600

    def body(h0_ref, a_ref, r_ref, w_ref, b_ref, o_ref):
        m = a_ref[...] - r_ref[...]
        o_ref[...] = jnp.maximum(
            h0_ref[...] + jnp.dot(m, w_ref[...], preferred_element_type=F32)
            + b_ref[...], 0.0)

    return pl.pallas_call(
        body,
        grid=(E // BE,),
        in_specs=[
            pl.BlockSpec((BE, D), lambda i: (i, 0)),
            pl.BlockSpec((BE, D), lambda i: (i, 0)),
            pl.BlockSpec((BE, D), lambda i: (i, 0)),
            pl.BlockSpec((D, D), lambda i: (0, 0)),
            pl.BlockSpec((1, D), lambda i: (0, 0)),
        ],
        out_specs=pl.BlockSpec((BE, D), lambda i: (i, 0)),
        out_shape=jax.ShapeDtypeStruct((E, D), F32),
    )(h0, arows, rrows, whT, bh)


def _tc_readout(x, mparts, woxT, womT, bo, n):
    D = x.shape[1]
    BN = 2000
    nb = n // BN

    def body(x_ref, p0_ref, p1_ref, wx_ref, wm_ref, b_ref, o_ref):
        mn = p0_ref[...] + p1_ref[...]
        iso = jnp.sum(mn, axis=1, keepdims=True) == 0.0
        mne = jnp.where(iso, x_ref[...], mn)
        o_ref[...] = jnp.maximum(
            jnp.dot(x_ref[...], wx_ref[...], preferred_element_type=F32)
            + jnp.dot(mne, wm_ref[...], preferred_element_type=F32)
            + b_ref[...], 0.0)

    return pl.pallas_call(
        body,
        grid=(nb,),
        in_specs=[
            pl.BlockSpec((BN, D), lambda i: (i, 0)),
            pl.BlockSpec((BN, D), lambda i: (i, 0)),
            pl.BlockSpec((BN, D), lambda i: (i + nb, 0)),
            pl.BlockSpec((D, D), lambda i: (0, 0)),
            pl.BlockSpec((D, D), lambda i: (0, 0)),
            pl.BlockSpec((1, D), lambda i: (0, 0)),
        ],
        out_specs=pl.BlockSpec((BN, D), lambda i: (i, 0)),
        out_shape=jax.ShapeDtypeStruct((n, D), F32),
    )(x, mparts, mparts, woxT, womT, bo)


def kernel(x, edge_index, edge_attr, rev_edge_index, W_i, b_i, W_h, b_h,
           W_o, b_o):
    n, df = x.shape
    hid = W_i.shape[0]
    depth = 5

    src = edge_index[0].astype(jnp.int32)
    dst = edge_index[1].astype(jnp.int32)
    rev = rev_edge_index.astype(jnp.int32)

    wixT = W_i[:, :df].T
    wieT = W_i[:, df:].T
    whT = W_h.T
    woxT = W_o[:, :df].T
    womT = W_o[:, df:].T
    bi = b_i.reshape(1, hid)
    bh = b_h.reshape(1, hid)
    bo = b_o.reshape(1, hid)
    zeros_n = jnp.zeros((n, hid), dtype=F32)

    xw = _tc_matmul(x, wixT, jnp.zeros((1, hid), dtype=F32))
    xws = _sc_gather(xw, src)
    h0, hc = _tc_input_layer(xws, edge_attr, wieT, bi)

    for _ in range(1, depth):
        aparts = _sc_scatter_add(hc, src, dst, zeros_n, gather_rows=True)
        a = _tc_psum(aparts, n)
        arows, rrows = _sc_dual_gather(a, hc, src, rev)
        hc = _tc_round(h0, arows, rrows, whT, bh)

    mparts = _sc_scatter_add(hc, src, dst, zeros_n, gather_rows=False)
    return _tc_readout(x, mparts, woxT, womT, bo, n)


# speedup vs baseline: 2.4776x; 1.0005x over previous
import functools

import jax
import jax.numpy as jnp
from jax import lax
from jax.experimental import pallas as pl
from jax.experimental.pallas import tpu as pltpu
from jax.experimental.pallas import tpu_sc as plsc

NC = 2
NS = 16
NW = NC * NS
CH = 128
K = 3

F32 = jnp.float32
BF16 = jnp.bfloat16


def _cdiv(a, b):
    return (a + b - 1) // b


def _sc_gather(table, idx):
    T, D = table.shape
    B = idx.shape[0]
    nchunk = B // CH
    bulk = (nchunk // NW) // K * K
    groups = bulk // K
    ntail = nchunk - bulk * NW
    assert ntail <= NW
    mesh = plsc.VectorSubcoreMesh(core_axis_name="c", subcore_axis_name="s")

    @functools.partial(
        pl.kernel,
        out_type=jax.ShapeDtypeStruct((B, D), F32),
        mesh=mesh,
        scratch_types=[
            tuple(pltpu.VMEM((CH,), jnp.int32) for _ in range(K)),
            tuple(pltpu.VMEM((CH, D), F32) for _ in range(K)),
            pltpu.SemaphoreType.DMA,
            pltpu.SemaphoreType.DMA,
            pltpu.SemaphoreType.DMA,
        ],
    )
    def k(table_hbm, idx_hbm, out_hbm, idx_bufs, row_bufs, sem_i, sem_g, sem_o):
        wid = lax.axis_index("s") * NC + lax.axis_index("c")

        @pl.loop(0, groups)
        def _(g):
            bases = [pl.multiple_of(((g * K + b) * NW + wid) * CH, CH)
                     for b in range(K)]
            ds = [pltpu.async_copy(idx_hbm.at[pl.ds(bases[b], CH)],
                                   idx_bufs[b], sem_i) for b in range(K)]
            for d in ds:
                d.wait()
            ds = [pltpu.async_copy(table_hbm.at[idx_bufs[b]], row_bufs[b],
                                   sem_g) for b in range(K)]
            for d in ds:
                d.wait()
            ds = [pltpu.async_copy(row_bufs[b], out_hbm.at[pl.ds(bases[b], CH)],
                                   sem_o) for b in range(K)]
            for d in ds:
                d.wait()

        @pl.when(wid < ntail)
        def _():
            base = pl.multiple_of((bulk * NW + wid) * CH, CH)
            pltpu.sync_copy(idx_hbm.at[pl.ds(base, CH)], idx_bufs[0])
            pltpu.async_copy(table_hbm.at[idx_bufs[0]], row_bufs[0],
                             sem_g).wait()
            pltpu.sync_copy(row_bufs[0], out_hbm.at[pl.ds(base, CH)])

    return k(table, idx)


def _sc_dual_gather(a, hc, src, rev):
    N, D = a.shape
    E = src.shape[0]
    nchunk = E // CH
    bulk = (nchunk // NW) // K * K
    groups = bulk // K
    ntail = nchunk - bulk * NW
    assert ntail <= NW
    mesh = plsc.VectorSubcoreMesh(core_axis_name="c", subcore_axis_name="s")

    @functools.partial(
        pl.kernel,
        out_type=(jax.ShapeDtypeStruct((E, D), F32),
                  jax.ShapeDtypeStruct((E, D), F32)),
        mesh=mesh,
        scratch_types=[
            tuple(pltpu.VMEM((CH,), jnp.int32) for _ in range(K)),
            tuple(pltpu.VMEM((CH,), jnp.int32) for _ in range(K)),
            tuple(pltpu.VMEM((CH, D), F32) for _ in range(K)),
            tuple(pltpu.VMEM((CH, D), F32) for _ in range(K)),
            pltpu.SemaphoreType.DMA,
            pltpu.SemaphoreType.DMA,
            pltpu.SemaphoreType.DMA,
        ],
    )
    def k(a_hbm, hc_hbm, src_hbm, rev_hbm, ar_hbm, rr_hbm,
          aidx_bufs, ridx_bufs, ar_bufs, rr_bufs, sem_i, sem_g, sem_o):
        wid = lax.axis_index("s") * NC + lax.axis_index("c")

        @pl.loop(0, groups)
        def _(g):
            bases = [pl.multiple_of(((g * K + b) * NW + wid) * CH, CH)
                     for b in range(K)]
            ds = []
            for b in range(K):
                ds.append(pltpu.async_copy(src_hbm.at[pl.ds(bases[b], CH)],
                                           aidx_bufs[b], sem_i))
                ds.append(pltpu.async_copy(rev_hbm.at[pl.ds(bases[b], CH)],
                                           ridx_bufs[b], sem_i))
            for d in ds:
                d.wait()
            ds = []
            for b in range(K):
                ds.append(pltpu.async_copy(a_hbm.at[aidx_bufs[b]], ar_bufs[b],
                                           sem_g))
                ds.append(pltpu.async_copy(hc_hbm.at[ridx_bufs[b]], rr_bufs[b],
                                           sem_g))
            for d in ds:
                d.wait()
            ds = []
            for b in range(K):
                ds.append(pltpu.async_copy(ar_bufs[b],
                                           ar_hbm.at[pl.ds(bases[b], CH)],
                                           sem_o))
                ds.append(pltpu.async_copy(rr_bufs[b],
                                           rr_hbm.at[pl.ds(bases[b], CH)],
                                           sem_o))
            for d in ds:
                d.wait()

        @pl.when(wid < ntail)
        def _():
            base = pl.multiple_of((bulk * NW + wid) * CH, CH)
            pltpu.sync_copy(src_hbm.at[pl.ds(base, CH)], aidx_bufs[0])
            pltpu.sync_copy(rev_hbm.at[pl.ds(base, CH)], ridx_bufs[0])
            da = pltpu.async_copy(a_hbm.at[aidx_bufs[0]], ar_bufs[0], sem_g)
            dr = pltpu.async_copy(hc_hbm.at[ridx_bufs[0]], rr_bufs[0], sem_g)
            da.wait()
            dr.wait()
            pltpu.sync_copy(ar_bufs[0], ar_hbm.at[pl.ds(base, CH)])
            pltpu.sync_copy(rr_bufs[0], rr_hbm.at[pl.ds(base, CH)])

    return k(a, hc, src, rev)


def _sc_scatter_add(rows_table, src, dst, zeros_n, gather_rows):
    T, D = rows_table.shape
    E = dst.shape[0]
    N = zeros_n.shape[0]
    nzc = 1000
    nz_tiles = N // nzc
    nchunk = E // CH
    per_core = nchunk // NC
    bulk = (per_core // NS) // K * K
    groups = bulk // K
    ntail = per_core - bulk * NS
    assert ntail <= NS
    mesh = plsc.VectorSubcoreMesh(core_axis_name="c", subcore_axis_name="s")

    @functools.partial(
        pl.kernel,
        out_type=jax.ShapeDtypeStruct((NC * N, D), F32),
        mesh=mesh,
        scratch_types=[
            tuple(pltpu.VMEM((CH,), jnp.int32) for _ in range(K)),
            tuple(pltpu.VMEM((CH,), jnp.int32) for _ in range(K)),
            tuple(pltpu.VMEM((CH, D), F32) for _ in range(K)),
            pltpu.VMEM_SHARED((N, D), F32),
            pltpu.SemaphoreType.DMA,
            pltpu.SemaphoreType.DMA,
            pltpu.SemaphoreType.DMA,
        ],
    )
    def k(table_hbm, src_hbm, dst_hbm, zeros_hbm, out_hbm,
          sidx_bufs, didx_bufs, row_bufs, acc_sh, sem_i, sem_g, sem_a):
        c = lax.axis_index("c")
        s = lax.axis_index("s")

        row0 = pl.multiple_of(s * nzc, 8)

        @pl.when(s < nz_tiles)
        def _():
            pltpu.sync_copy(zeros_hbm.at[pl.ds(row0, nzc)],
                            acc_sh.at[pl.ds(row0, nzc)])

        plsc.subcore_barrier()

        @pl.loop(0, groups)
        def _(g):
            bases = [pl.multiple_of(
                (c * per_core + (g * K + b) * NS + s) * CH, CH)
                for b in range(K)]
            ds = []
            for b in range(K):
                ds.append(pltpu.async_copy(dst_hbm.at[pl.ds(bases[b], CH)],
                                           didx_bufs[b], sem_i))
                if gather_rows:
                    ds.append(pltpu.async_copy(src_hbm.at[pl.ds(bases[b], CH)],
                                               sidx_bufs[b], sem_i))
            for d in ds:
                d.wait()
            ds = []
            for b in range(K):
                if gather_rows:
                    ds.append(pltpu.async_copy(table_hbm.at[sidx_bufs[b]],
                                               row_bufs[b], sem_g))
                else:
                    ds.append(pltpu.async_copy(
                        table_hbm.at[pl.ds(bases[b], CH)], row_bufs[b], sem_g))
            for d in ds:
                d.wait()
            ds = [pltpu.async_copy(row_bufs[b], acc_sh.at[didx_bufs[b]],
                                   sem_a, add=True) for b in range(K)]
            for d in ds:
                d.wait()

        @pl.when(s < ntail)
        def _():
            base = pl.multiple_of((c * per_core + bulk * NS + s) * CH, CH)
            pltpu.sync_copy(dst_hbm.at[pl.ds(base, CH)], didx_bufs[0])
            if gather_rows:
                pltpu.sync_copy(src_hbm.at[pl.ds(base, CH)], sidx_bufs[0])
                pltpu.async_copy(table_hbm.at[sidx_bufs[0]], row_bufs[0],
                                 sem_g).wait()
            else:
                pltpu.sync_copy(table_hbm.at[pl.ds(base, CH)], row_bufs[0])
            pltpu.sync_copy(row_bufs[0], acc_sh.at[didx_bufs[0]], add=True)

        plsc.subcore_barrier()

        @pl.when(s < nz_tiles)
        def _():
            out0 = pl.multiple_of(c * N + s * nzc, 8)
            pltpu.sync_copy(acc_sh.at[pl.ds(row0, nzc)],
                            out_hbm.at[pl.ds(out0, nzc)])

    return k(rows_table, src, dst, zeros_n)


def _tc_matmul(a, w, b):
    M, Kd = a.shape
    Dout = w.shape[1]
    BM = 2000 if M % 2000 == 0 else M

    def body(a_ref, w_ref, b_ref, o_ref):
        o_ref[...] = jnp.dot(a_ref[...], w_ref[...],
                             preferred_element_type=F32) + b_ref[...]

    return pl.pallas_call(
        body,
        grid=(M // BM,),
        in_specs=[
            pl.BlockSpec((BM, Kd), lambda i: (i, 0)),
            pl.BlockSpec((Kd, Dout), lambda i: (0, 0)),
            pl.BlockSpec((1, Dout), lambda i: (0, 0)),
        ],
        out_specs=pl.BlockSpec((BM, Dout), lambda i: (i, 0)),
        out_shape=jax.ShapeDtypeStruct((M, Dout), F32),
    )(a, w, b)


def _tc_input_layer(xws, ea, wieT, bi):
    E, D = xws.shape
    DE = ea.shape[1]
    BE = 1600

    def body(xws_ref, ea_ref, w_ref, b_ref, h0_ref, hc_ref):
        h0 = xws_ref[...] + jnp.dot(ea_ref[...], w_ref[...],
                                    preferred_element_type=F32) + b_ref[...]
        h0_ref[...] = h0
        hc_ref[...] = jnp.maximum(h0, 0.0)

    return pl.pallas_call(
        body,
        grid=(E // BE,),
        in_specs=[
            pl.BlockSpec((BE, D), lambda i: (i, 0)),
            pl.BlockSpec((BE, DE), lambda i: (i, 0)),
            pl.BlockSpec((DE, D), lambda i: (0, 0)),
            pl.BlockSpec((1, D), lambda i: (0, 0)),
        ],
        out_specs=[
            pl.BlockSpec((BE, D), lambda i: (i, 0)),
            pl.BlockSpec((BE, D), lambda i: (i, 0)),
        ],
        out_shape=[
            jax.ShapeDtypeStruct((E, D), F32),
            jax.ShapeDtypeStruct((E, D), F32),
        ],
    )(xws, ea, wieT, bi)


def _tc_psum(p, n):
    D = p.shape[1]
    BN = 2000
    nb = n // BN

    def body(a_ref, b_ref, o_ref):
        o_ref[...] = a_ref[...] + b_ref[...]

    return pl.pallas_call(
        body,
        grid=(nb,),
        in_specs=[
            pl.BlockSpec((BN, D), lambda i: (i, 0)),
            pl.BlockSpec((BN, D), lambda i: (i + nb, 0)),
        ],
        out_specs=pl.BlockSpec((BN, D), lambda i: (i, 0)),
        out_shape=jax.ShapeDtypeStruct((n, D), F32),
    )(p, p)


def _tc_round(h0, arows, rrows, whT, bh):
    E, D = h0.shape
    BE = 1600

    def body(h0_ref, a_ref, r_ref, w_ref, b_ref, o_ref):
        m = a_ref[...] - r_ref[...]
        o_ref[...] = jnp.maximum(
            h0_ref[...] + jnp.dot(m, w_ref[...], preferred_element_type=F32)
            + b_ref[...], 0.0)

    return pl.pallas_call(
        body,
        grid=(E // BE,),
        in_specs=[
            pl.BlockSpec((BE, D), lambda i: (i, 0)),
            pl.BlockSpec((BE, D), lambda i: (i, 0)),
            pl.BlockSpec((BE, D), lambda i: (i, 0)),
            pl.BlockSpec((D, D), lambda i: (0, 0)),
            pl.BlockSpec((1, D), lambda i: (0, 0)),
        ],
        out_specs=pl.BlockSpec((BE, D), lambda i: (i, 0)),
        out_shape=jax.ShapeDtypeStruct((E, D), F32),
    )(h0, arows, rrows, whT, bh)


def _tc_readout(x, mparts, woxT, womT, bo, n):
    D = x.shape[1]
    BN = 2000
    nb = n // BN

    def body(x_ref, p0_ref, p1_ref, wx_ref, wm_ref, b_ref, o_ref):
        mn = p0_ref[...] + p1_ref[...]
        iso = jnp.sum(mn, axis=1, keepdims=True) == 0.0
        mne = jnp.where(iso, x_ref[...], mn)
        o_ref[...] = jnp.maximum(
            jnp.dot(x_ref[...], wx_ref[...], preferred_element_type=F32)
            + jnp.dot(mne, wm_ref[...], preferred_element_type=F32)
            + b_ref[...], 0.0)

    return pl.pallas_call(
        body,
        grid=(nb,),
        in_specs=[
            pl.BlockSpec((BN, D), lambda i: (i, 0)),
            pl.BlockSpec((BN, D), lambda i: (i, 0)),
            pl.BlockSpec((BN, D), lambda i: (i + nb, 0)),
            pl.BlockSpec((D, D), lambda i: (0, 0)),
            pl.BlockSpec((D, D), lambda i: (0, 0)),
            pl.BlockSpec((1, D), lambda i: (0, 0)),
        ],
        out_specs=pl.BlockSpec((BN, D), lambda i: (i, 0)),
        out_shape=jax.ShapeDtypeStruct((n, D), F32),
    )(x, mparts, mparts, woxT, womT, bo)


def kernel(x, edge_index, edge_attr, rev_edge_index, W_i, b_i, W_h, b_h,
           W_o, b_o):
    n, df = x.shape
    hid = W_i.shape[0]
    depth = 5

    src = edge_index[0].astype(jnp.int32)
    dst = edge_index[1].astype(jnp.int32)
    rev = rev_edge_index.astype(jnp.int32)

    wixT = W_i[:, :df].T
    wieT = W_i[:, df:].T
    whT = W_h.T
    woxT = W_o[:, :df].T
    womT = W_o[:, df:].T
    bi = b_i.reshape(1, hid)
    bh = b_h.reshape(1, hid)
    bo = b_o.reshape(1, hid)
    zeros_n = jnp.zeros((n, hid), dtype=F32)

    xw = _tc_matmul(x, wixT, jnp.zeros((1, hid), dtype=F32))
    xws = _sc_gather(xw, src)
    h0, hc = _tc_input_layer(xws, edge_attr, wieT, bi)

    for _ in range(1, depth):
        aparts = _sc_scatter_add(hc, src, dst, zeros_n, gather_rows=True)
        a = _tc_psum(aparts, n)
        arows, rrows = _sc_dual_gather(a, hc, src, rev)
        hc = _tc_round(h0, arows, rrows, whT, bh)

    mparts = _sc_scatter_add(hc, src, dst, zeros_n, gather_rows=False)
    return _tc_readout(x, mparts, woxT, womT, bo, n)
